# baseline (speedup 1.0000x reference)
import jax
import jax.numpy as jnp
from jax import lax
from jax.experimental import pallas as pl
from jax.experimental.pallas import tpu as pltpu
from jax.experimental.pallas import tpu_sc as plsc

N_NODES = 10000
N_EDGES = 320000
D_IN = 128
D_HID = 256

NC = 2
NS = 16
L = 16
C = 128
N_PAD = 10240
HN = 5120
ACC_ROWS = 5376
ZROWS = ACC_ROWS // NS
WROWS = HN // NS
DUMMY = HN
DST_PAD = 2 * HN

J2 = 158
E_PAD2 = NS * J2 * C


def _chunk_pad(a, e_pad, fill, lead_shape):
    pad = e_pad - N_EDGES
    ap = jnp.concatenate([a, jnp.full((pad,), fill, jnp.int32)])
    return ap.reshape(lead_shape)


def _phase_dst(dst):
    outs = []
    for p in range(2):
        lo = p * HN
        inr = (dst >= lo) & (dst < lo + HN)
        outs.append(jnp.where(inr, dst - lo, DUMMY))
    return jnp.stack(outs)


def _make_sc_agg2():
    dh = D_HID // 2
    mesh = plsc.VectorSubcoreMesh(core_axis_name="c", subcore_axis_name="s")
    out_types = (jax.ShapeDtypeStruct((2, N_PAD, dh), jnp.float32),)
    scratch = [
        pltpu.VMEM((C,), jnp.int32),
        pltpu.VMEM((C,), jnp.int32),
        pltpu.VMEM((C, dh), jnp.float32),
        pltpu.VMEM_SHARED((ACC_ROWS, dh), jnp.float32),
        pltpu.SemaphoreType.DMA,
    ]

    def body(h2d, srcs, dsts, zrow, out_agg,
             src_c, dst_c, rows0, acc_sh, sem0):
        c = lax.axis_index("c")
        s = lax.axis_index("s")
        for p in range(2):
            z0 = s * ZROWS
            pltpu.sync_copy(zrow, rows0)
            pltpu.sync_copy(rows0, acc_sh.at[pl.ds(z0, C)])
            pltpu.sync_copy(rows0, acc_sh.at[pl.ds(z0 + C, C)])
            pltpu.sync_copy(rows0.at[pl.ds(0, 80)],
                            acc_sh.at[pl.ds(z0 + 2 * C, 80)])
            plsc.subcore_barrier()

            def step(j, _):
                pltpu.sync_copy(srcs.at[c, s, j], src_c)
                pltpu.sync_copy(dsts.at[p, s, j], dst_c)
                pltpu.async_copy(h2d.at[src_c], rows0, sem0).wait()
                pltpu.sync_copy(rows0, acc_sh.at[dst_c], add=True)
                return 0
            lax.fori_loop(0, J2, step, 0)
            plsc.subcore_barrier()
            w0 = s * WROWS
            for k, nr in ((0, C), (C, C), (2 * C, 64)):
                pltpu.sync_copy(acc_sh.at[pl.ds(w0 + k, nr)],
                                rows0.at[pl.ds(0, nr)])
                pltpu.sync_copy(rows0.at[pl.ds(0, nr)],
                                out_agg.at[c, pl.ds(p * HN + w0 + k, nr)])

    return pl.kernel(body, out_type=out_types, mesh=mesh,
                     scratch_types=scratch)


def _make_tc_layer1():
    R = 1024
    nb = N_PAD // R
    row = lambda i: (i, 0)
    full = lambda i: (0, 0)
    in_specs = [
        pl.BlockSpec((R, D_IN), row),
        pl.BlockSpec((1, R, D_IN), lambda i: (0, i, 0)),
        pl.BlockSpec((1, R, D_IN), lambda i: (1, i, 0)),
        pl.BlockSpec((D_IN, D_HID), full),
        pl.BlockSpec((D_IN, D_HID), full),
        pl.BlockSpec((1, D_HID), full),
    ]

    def body(xb, ag, dg, ws, wn, b, o):
        invd = 1.0 / jnp.maximum(dg[0][:, 0:1], 1.0)
        mean = ag[0] * invd
        acc = jnp.dot(xb[...], ws[...], preferred_element_type=jnp.float32)
        acc += jnp.dot(mean, wn[...], preferred_element_type=jnp.float32)
        acc += b[...]
        acc = jnp.maximum(acc, 0.0)
        o[0] = acc[:, :128]
        o[1] = acc[:, 128:]

    return pl.pallas_call(
        body, grid=(nb,), in_specs=in_specs,
        out_specs=pl.BlockSpec((2, R, 128), lambda i: (0, i, 0)),
        out_shape=jax.ShapeDtypeStruct((2, N_PAD, 128), jnp.float32))


def _make_tc_layer23(relu, stacked_out):
    dh = D_HID // 2
    R = 1024
    nb = N_PAD // R
    row_l = lambda i: (i, 0)
    row_r = lambda i: (i + nb, 0)
    full = lambda i: (0, 0)
    in_specs = [
        pl.BlockSpec((R, dh), row_l),
        pl.BlockSpec((R, dh), row_r),
        pl.BlockSpec((R, dh), row_l),
        pl.BlockSpec((R, dh), row_r),
        pl.BlockSpec((1, R, D_IN), lambda i: (1, i, 0)),
        pl.BlockSpec((D_HID, D_HID), full),
        pl.BlockSpec((D_HID, D_HID), full),
        pl.BlockSpec((1, D_HID), full),
    ]
    if stacked_out:
        out_spec = pl.BlockSpec((2, R, 128), lambda i: (0, i, 0))
        out_shape = jax.ShapeDtypeStruct((2, N_PAD, 128), jnp.float32)
    else:
        out_spec = pl.BlockSpec((R, D_HID), row_l)
        out_shape = jax.ShapeDtypeStruct((N_PAD, D_HID), jnp.float32)

    def body(h_l, h_r, a_l, a_r, dg, ws, wn, b, o):
        invd = 1.0 / jnp.maximum(dg[0][:, 0:1], 1.0)
        acc = jnp.dot(h_l[...], ws[:dh, :], preferred_element_type=jnp.float32)
        acc += jnp.dot(h_r[...], ws[dh:, :], preferred_element_type=jnp.float32)
        acc += jnp.dot(a_l[...] * invd, wn[:dh, :],
                       preferred_element_type=jnp.float32)
        acc += jnp.dot(a_r[...] * invd, wn[dh:, :],
                       preferred_element_type=jnp.float32)
        acc += b[...]
        if relu:
            acc = jnp.maximum(acc, 0.0)
        if stacked_out:
            o[0] = acc[:, :128]
            o[1] = acc[:, 128:]
        else:
            o[...] = acc

    return pl.pallas_call(
        body, grid=(nb,), in_specs=in_specs, out_specs=out_spec,
        out_shape=out_shape)


_sc_agg2 = _make_sc_agg2()
_tc_l1 = _make_tc_layer1()
_tc_l2 = _make_tc_layer23(relu=True, stacked_out=True)
_tc_l3 = _make_tc_layer23(relu=False, stacked_out=False)


def kernel(x, edge_index, W_self1, W_neigh1, b1, W_self2, W_neigh2, b2,
           W_self3, W_neigh3, b3):
    src = edge_index[0].astype(jnp.int32)
    dst = edge_index[1].astype(jnp.int32)
    src2 = _chunk_pad(src, E_PAD2, 0, (NS, J2, C))
    srcs2 = jnp.stack([src2, src2 + N_PAD])
    srcs1 = jnp.stack([src2, jnp.full_like(src2, N_PAD)])
    dst2 = _chunk_pad(dst, E_PAD2, DST_PAD, (NS, J2, C))
    dsts2 = _phase_dst(dst2)

    zrow = jnp.zeros((C, D_IN), jnp.float32)

    xp = jnp.pad(x, ((0, N_PAD - N_NODES), (0, 0)))
    onehot = jnp.zeros((N_PAD, D_IN), jnp.float32).at[0, 0].set(1.0)
    x_aug = jnp.concatenate([xp, onehot], axis=0)
    b1r = b1.reshape(1, D_HID)
    b2r = b2.reshape(1, D_HID)
    b3r = b3.reshape(1, D_HID)

    (agg1,) = _sc_agg2(x_aug, srcs1, dsts2, zrow)
    h2 = _tc_l1(xp, agg1, agg1, W_self1, W_neigh1, b1r)
    h2d = h2.reshape(2 * N_PAD, 128)

    (agg2,) = _sc_agg2(h2d, srcs2, dsts2, zrow)
    a2 = agg2.reshape(2 * N_PAD, 128)
    h3 = _tc_l2(h2d, h2d, a2, a2, agg1, W_self2, W_neigh2, b2r)
    h3d = h3.reshape(2 * N_PAD, 128)

    (agg3,) = _sc_agg2(h3d, srcs2, dsts2, zrow)
    a3 = agg3.reshape(2 * N_PAD, 128)
    out = _tc_l3(h3d, h3d, a3, a3, agg1, W_self3, W_neigh3, b3r)
    return out[:N_NODES]
